# Initial kernel scaffold; baseline (speedup 1.0000x reference)
import jax
import jax.numpy as jnp
from jax.experimental import pallas as pl


def kernel(x, adj, W0, b0, W1, b1, W2, b2, W3, b3):
    raise NotImplementedError("write your pallas kernel here")


# speedup vs baseline: 1.0533x; 1.0533x over previous
import functools

import jax
import jax.numpy as jnp
from jax.experimental import pallas as pl
from jax.experimental.pallas import tpu as pltpu

N = 10000
F = 256
C = 128

_DOT = functools.partial(
    jax.lax.dot_general,
    dimension_numbers=(((1,), (0,)), ((), ())),
    preferred_element_type=jnp.float32,
)


def _sup0_body(x_ref, w_ref, out_ref):
    out_ref[...] = _DOT(
        x_ref[...].astype(jnp.bfloat16), w_ref[...]
    ).astype(jnp.bfloat16)


def _layer1_body(adj_ref, sup_ref, b_ref, w_ref, adjbf_ref, x1_ref, sup2_ref):
    a = adj_ref[...].astype(jnp.bfloat16)
    adjbf_ref[...] = a
    acc = _DOT(a, sup_ref[...])
    h = jnp.maximum(acc + b_ref[...], 0.0)
    x1_ref[...] = h
    sup2_ref[...] = _DOT(h.astype(jnp.bfloat16), w_ref[...]).astype(jnp.bfloat16)


def _mid_body(adj_ref, sup_ref, b_ref, w_ref, supn_ref):
    acc = _DOT(adj_ref[...], sup_ref[...])
    h = jnp.maximum(acc + b_ref[...], 0.0)
    supn_ref[...] = _DOT(h.astype(jnp.bfloat16), w_ref[...]).astype(jnp.bfloat16)


def _res_body(adj_ref, sup_ref, b_ref, w_ref, res_ref, supn_ref):
    acc = _DOT(adj_ref[...], sup_ref[...])
    h = jnp.maximum(acc + b_ref[...], 0.0) + res_ref[...]
    supn_ref[...] = _DOT(h.astype(jnp.bfloat16), w_ref[...]).astype(jnp.bfloat16)


def _final_body(adj_ref, sup_ref, b_ref, out_ref):
    z = _DOT(adj_ref[...], sup_ref[...]) + b_ref[...]
    m = jnp.max(z, axis=1, keepdims=True)
    lse = jnp.log(jnp.sum(jnp.exp(z - m), axis=1, keepdims=True)) + m
    out_ref[...] = z - lse


def _row_spec(bm, cols):
    return pl.BlockSpec((bm, cols), lambda i: (i, 0))


def _full_spec(rows, cols):
    return pl.BlockSpec((rows, cols), lambda i: (0, 0))


_PARAMS = pltpu.CompilerParams(dimension_semantics=("arbitrary",))


def kernel(x, adj, W0, b0, W1, b1, W2, b2, W3, b3):
    w0 = W0.astype(jnp.bfloat16)
    w1 = W1.astype(jnp.bfloat16)
    w2 = W2.astype(jnp.bfloat16)
    w3 = W3.astype(jnp.bfloat16)
    b0r = b0.reshape(1, F)
    b1r = b1.reshape(1, F)
    b2r = b2.reshape(1, F)
    b3r = b3.reshape(1, C)

    sup1 = pl.pallas_call(
        _sup0_body,
        grid=(5,),
        in_specs=[_row_spec(2000, F), _full_spec(F, F)],
        out_specs=_row_spec(2000, F),
        out_shape=jax.ShapeDtypeStruct((N, F), jnp.bfloat16),
        compiler_params=_PARAMS,
    )(x, w0)

    adj_bf, x1, sup2 = pl.pallas_call(
        _layer1_body,
        grid=(125,),
        in_specs=[
            _row_spec(80, N),
            _full_spec(N, F),
            _full_spec(1, F),
            _full_spec(F, F),
        ],
        out_specs=(
            _row_spec(80, N),
            _row_spec(80, F),
            _row_spec(80, F),
        ),
        out_shape=(
            jax.ShapeDtypeStruct((N, N), jnp.bfloat16),
            jax.ShapeDtypeStruct((N, F), jnp.float32),
            jax.ShapeDtypeStruct((N, F), jnp.bfloat16),
        ),
        compiler_params=_PARAMS,
    )(adj, sup1, b0r, w1)

    sup3 = pl.pallas_call(
        _mid_body,
        grid=(25,),
        in_specs=[
            _row_spec(400, N),
            _full_spec(N, F),
            _full_spec(1, F),
            _full_spec(F, F),
        ],
        out_specs=_row_spec(400, F),
        out_shape=jax.ShapeDtypeStruct((N, F), jnp.bfloat16),
        compiler_params=_PARAMS,
    )(adj_bf, sup2, b1r, w2)

    sup4 = pl.pallas_call(
        _res_body,
        grid=(25,),
        in_specs=[
            _row_spec(400, N),
            _full_spec(N, F),
            _full_spec(1, F),
            _full_spec(F, C),
            _row_spec(400, F),
        ],
        out_specs=_row_spec(400, C),
        out_shape=jax.ShapeDtypeStruct((N, C), jnp.bfloat16),
        compiler_params=_PARAMS,
    )(adj_bf, sup3, b2r, w3, x1)

    out = pl.pallas_call(
        _final_body,
        grid=(25,),
        in_specs=[
            _row_spec(400, N),
            _full_spec(N, C),
            _full_spec(1, C),
        ],
        out_specs=_row_spec(400, C),
        out_shape=jax.ShapeDtypeStruct((N, C), jnp.float32),
        compiler_params=_PARAMS,
    )(adj_bf, sup4, b3r)

    return out


# speedup vs baseline: 1.1923x; 1.1320x over previous
import functools

import jax
import jax.numpy as jnp
from jax.experimental import pallas as pl
from jax.experimental.pallas import tpu as pltpu

N = 10000
F = 256
C = 128

_DOT = functools.partial(
    jax.lax.dot_general,
    dimension_numbers=(((1,), (0,)), ((), ())),
    preferred_element_type=jnp.float32,
)


def _sup0_body(x_ref, w_ref, out_ref):
    out_ref[...] = _DOT(
        x_ref[...].astype(jnp.bfloat16), w_ref[...]
    ).astype(jnp.bfloat16)


def _layer1_body(adj_ref, sup_ref, b_ref, w_ref, adjbf_ref, x1_ref, sup2_ref):
    a = adj_ref[...].astype(jnp.bfloat16)
    adjbf_ref[...] = a
    acc = _DOT(a, sup_ref[...])
    h = jnp.maximum(acc + b_ref[...], 0.0)
    x1_ref[...] = h.astype(jnp.bfloat16)
    sup2_ref[...] = _DOT(h.astype(jnp.bfloat16), w_ref[...]).astype(jnp.bfloat16)


def _mid_body(adj_ref, sup_ref, b_ref, w_ref, supn_ref):
    acc = _DOT(adj_ref[...], sup_ref[...])
    h = jnp.maximum(acc + b_ref[...], 0.0)
    supn_ref[...] = _DOT(h.astype(jnp.bfloat16), w_ref[...]).astype(jnp.bfloat16)


def _res_body(adj_ref, sup_ref, b_ref, w_ref, res_ref, supn_ref):
    acc = _DOT(adj_ref[...], sup_ref[...])
    h = jnp.maximum(acc + b_ref[...], 0.0) + res_ref[...].astype(jnp.float32)
    supn_ref[...] = _DOT(h.astype(jnp.bfloat16), w_ref[...]).astype(jnp.bfloat16)


def _final_body(adj_ref, sup_ref, b_ref, out_ref):
    z = _DOT(adj_ref[...], sup_ref[...]) + b_ref[...]
    m = jnp.max(z, axis=1, keepdims=True)
    lse = jnp.log(jnp.sum(jnp.exp(z - m), axis=1, keepdims=True)) + m
    out_ref[...] = z - lse


def _row_spec(bm, cols):
    return pl.BlockSpec((bm, cols), lambda i: (i, 0))


def _full_spec(rows, cols):
    return pl.BlockSpec((rows, cols), lambda i: (0, 0))


_PARAMS = pltpu.CompilerParams(dimension_semantics=("arbitrary",))


def kernel(x, adj, W0, b0, W1, b1, W2, b2, W3, b3):
    w0 = W0.astype(jnp.bfloat16)
    w1 = W1.astype(jnp.bfloat16)
    w2 = W2.astype(jnp.bfloat16)
    w3 = W3.astype(jnp.bfloat16)
    b0r = b0.reshape(1, F)
    b1r = b1.reshape(1, F)
    b2r = b2.reshape(1, F)
    b3r = b3.reshape(1, C)

    sup1 = pl.pallas_call(
        _sup0_body,
        grid=(5,),
        in_specs=[_row_spec(2000, F), _full_spec(F, F)],
        out_specs=_row_spec(2000, F),
        out_shape=jax.ShapeDtypeStruct((N, F), jnp.bfloat16),
        compiler_params=_PARAMS,
    )(x, w0)

    adj_bf, x1, sup2 = pl.pallas_call(
        _layer1_body,
        grid=(25,),
        in_specs=[
            _row_spec(400, N),
            _full_spec(N, F),
            _full_spec(1, F),
            _full_spec(F, F),
        ],
        out_specs=(
            _row_spec(400, N),
            _row_spec(400, F),
            _row_spec(400, F),
        ),
        out_shape=(
            jax.ShapeDtypeStruct((N, N), jnp.bfloat16),
            jax.ShapeDtypeStruct((N, F), jnp.bfloat16),
            jax.ShapeDtypeStruct((N, F), jnp.bfloat16),
        ),
        compiler_params=_PARAMS,
    )(adj, sup1, b0r, w1)

    sup3 = pl.pallas_call(
        _mid_body,
        grid=(25,),
        in_specs=[
            _row_spec(400, N),
            _full_spec(N, F),
            _full_spec(1, F),
            _full_spec(F, F),
        ],
        out_specs=_row_spec(400, F),
        out_shape=jax.ShapeDtypeStruct((N, F), jnp.bfloat16),
        compiler_params=_PARAMS,
    )(adj_bf, sup2, b1r, w2)

    sup4 = pl.pallas_call(
        _res_body,
        grid=(25,),
        in_specs=[
            _row_spec(400, N),
            _full_spec(N, F),
            _full_spec(1, F),
            _full_spec(F, C),
            _row_spec(400, F),
        ],
        out_specs=_row_spec(400, C),
        out_shape=jax.ShapeDtypeStruct((N, C), jnp.bfloat16),
        compiler_params=_PARAMS,
    )(adj_bf, sup3, b2r, w3, x1)

    out = pl.pallas_call(
        _final_body,
        grid=(25,),
        in_specs=[
            _row_spec(400, N),
            _full_spec(N, C),
            _full_spec(1, C),
        ],
        out_specs=_row_spec(400, C),
        out_shape=jax.ShapeDtypeStruct((N, C), jnp.float32),
        compiler_params=_PARAMS,
    )(adj_bf, sup4, b3r)

    return out
